# baseline (speedup 1.0000x reference)
import functools

import jax
import jax.numpy as jnp
from jax import lax
from jax.experimental import pallas as pl
from jax.experimental.pallas import tpu as pltpu
from jax.experimental.pallas import tpu_sc as plsc

_D = 64
_BATCH = 4096
_FIELDS = 26
_NC, _NS = 2, 16
_NW = _NC * _NS
_WBLK = _BATCH // _NW
_NG = _WBLK // 16

_mesh = plsc.VectorSubcoreMesh(core_axis_name="c", subcore_axis_name="s")


@functools.partial(
    pl.kernel,
    out_type=jax.ShapeDtypeStruct((_FIELDS, _D, _BATCH), jnp.float32),
    mesh=_mesh,
    scratch_types=[
        pltpu.VMEM((_FIELDS, _WBLK), jnp.int32),
        pltpu.VMEM((_FIELDS, _WBLK), jnp.int32),
        pltpu.VMEM((_FIELDS, _WBLK), jnp.int32),
        pltpu.VMEM((2, _WBLK, 2 * _D), jnp.float32),
        pltpu.VMEM((2, _D, _WBLK), jnp.float32),
        pltpu.SemaphoreType.DMA((2,)),
        pltpu.SemaphoreType.DMA((2,)),
    ],
    compiler_params=pltpu.CompilerParams(needs_layout_passes=False),
)
def _lookup(xt_hbm, table2_hbm, out_hbm, idx_v, q_v, h_v, rows_v, ot_v, gsem, osem):
    wid = lax.axis_index("s") * _NC + lax.axis_index("c")
    base = wid * _WBLK
    pltpu.sync_copy(xt_hbm.at[:, pl.ds(base, _WBLK)], idx_v)

    for f in range(_FIELDS):
        for k in range(_NG):
            sl = pl.ds(16 * k, 16)
            v = idx_v[f, sl]
            q_v[f, sl] = lax.shift_right_logical(v, 1)
            h_v[f, sl] = lax.shift_left(v & 1, 6)

    def gather(f, b):
        return pltpu.make_async_copy(
            table2_hbm.at[q_v.at[f]], rows_v.at[b], gsem.at[b]
        )

    def write(f, b):
        return pltpu.make_async_copy(
            ot_v.at[b], out_hbm.at[f, :, pl.ds(base, _WBLK)], osem.at[b]
        )

    lane = lax.iota(jnp.int32, 16)

    def select_transpose(f, b):
        rows = rows_v.at[b]
        ot = ot_v.at[b]

        @pl.loop(0, _D, unroll=8)
        def _d(d):
            for g in range(_NG):
                sl = pl.ds(16 * g, 16)
                cols = h_v[f, sl] + d
                vals = plsc.load_gather(rows, [lane + 16 * g, cols])
                ot[d, sl] = vals

    gather(0, 0).start()
    gather(1, 1).start()
    for f in (0, 1):
        b = f % 2
        gather(f, b).wait()
        select_transpose(f, b)
        gather(f + 2, b).start()
        write(f, b).start()

    @pl.loop(2, 24, step=2)
    def _steady(f):
        for b in (0, 1):
            fb = f + b
            gather(fb, b).wait()
            write(fb - 2, b).wait()
            select_transpose(fb, b)
            gather(fb + 2, b).start()
            write(fb, b).start()

    for fb in (24, 25):
        b = fb % 2
        gather(fb, b).wait()
        write(fb - 2, b).wait()
        select_transpose(fb, b)
        write(fb, b).start()
    write(24, 0).wait()
    write(25, 1).wait()


def kernel(x, word_mat):
    xt = x.T
    table2 = word_mat.reshape(500000, 2 * _D)
    o3 = _lookup(xt, table2)
    return o3.transpose(2, 0, 1)


# speedup vs baseline: 1.1954x; 1.1954x over previous
import functools

import jax
import jax.numpy as jnp
from jax import lax
from jax.experimental import pallas as pl
from jax.experimental.pallas import tpu as pltpu
from jax.experimental.pallas import tpu_sc as plsc

_D = 64
_BATCH = 4096
_FIELDS = 26
_NC, _NS = 2, 16
_NW = _NC * _NS
_WBLK = _BATCH // _NW
_NBUF = 4
_LOOKAHEAD = _NBUF - 1

_mesh = plsc.VectorSubcoreMesh(core_axis_name="c", subcore_axis_name="s")


@functools.partial(
    pl.kernel,
    out_type=jax.ShapeDtypeStruct((_FIELDS, _BATCH, 2 * _D), jnp.float32),
    mesh=_mesh,
    scratch_types=[
        pltpu.VMEM((_FIELDS, _WBLK), jnp.int32),
        pltpu.VMEM((_FIELDS, _WBLK), jnp.int32),
        pltpu.VMEM((_NBUF, _WBLK, 2 * _D), jnp.float32),
        pltpu.SemaphoreType.DMA((_NBUF,)),
        pltpu.SemaphoreType.DMA((_NBUF,)),
    ],
)
def _gather_rows(xt_hbm, table2_hbm, out_hbm, idx_v, q_v, rows_v, gsem, osem):
    wid = lax.axis_index("s") * _NC + lax.axis_index("c")
    base = wid * _WBLK
    pltpu.sync_copy(xt_hbm.at[:, pl.ds(base, _WBLK)], idx_v)

    for f in range(_FIELDS):
        for k in range(_WBLK // 16):
            sl = pl.ds(16 * k, 16)
            q_v[f, sl] = lax.shift_right_logical(idx_v[f, sl], 1)

    def gather(f):
        b = f % _NBUF
        return pltpu.make_async_copy(
            table2_hbm.at[q_v.at[f]], rows_v.at[b], gsem.at[b]
        )

    def write(f):
        b = f % _NBUF
        return pltpu.make_async_copy(
            rows_v.at[b], out_hbm.at[f, pl.ds(base, _WBLK)], osem.at[b]
        )

    for f in range(_LOOKAHEAD):
        gather(f).start()
    for f in range(_FIELDS):
        gather(f).wait()
        write(f).start()
        if f + _LOOKAHEAD < _FIELDS:
            if f >= 1:
                write(f - 1).wait()
            gather(f + _LOOKAHEAD).start()
    for f in range(max(0, _FIELDS - _LOOKAHEAD - 1), _FIELDS):
        write(f).wait()


def kernel(x, word_mat):
    xt = x.T
    table2 = word_mat.reshape(500000, 2 * _D)
    pairs = _gather_rows(xt, table2)
    odd = (xt & 1)[:, :, None] == 1
    out_t = jnp.where(odd, pairs[:, :, _D:], pairs[:, :, :_D])
    return out_t.transpose(1, 0, 2)
